# Initial kernel scaffold; baseline (speedup 1.0000x reference)
import jax
import jax.numpy as jnp
from jax.experimental import pallas as pl


def kernel(data, emb, lin_w, att_i, att_j, att_em_i, att_em_j, gl_bias, bn1_g, bn1_b, bno_g, bno_b, w1, b1, bn2_g, bn2_b, w2, b2):
    raise NotImplementedError("write your pallas kernel here")


# speedup vs baseline: 266.8253x; 266.8253x over previous
import jax
import jax.numpy as jnp
from jax.experimental import pallas as pl
from jax.experimental.pallas import tpu as pltpu

NODE_NUM = 512
DIM = 128
INPUT_DIM = 5
TOPK = 30
BATCH = 32
INTER = 256
EPS = 1e-5
NEG_INF = float("-inf")


def _mask_kernel(emb_ref, mask_ref):
    w = emb_ref[...]
    g = jnp.dot(w, w.T, preferred_element_type=jnp.float32)
    nrm = jnp.sqrt(jnp.sum(w * w, axis=1, keepdims=True))
    cos = g / (nrm * nrm.T)
    col = jax.lax.broadcasted_iota(jnp.int32, (NODE_NUM, NODE_NUM), 1)
    row = jax.lax.broadcasted_iota(jnp.int32, (NODE_NUM, NODE_NUM), 0)

    cur = cos
    msk = jnp.zeros((NODE_NUM, NODE_NUM), jnp.bool_)
    for _ in range(TOPK):
        m = jnp.max(cur, axis=1, keepdims=True)
        eq = cur == m
        sel = jnp.min(jnp.where(eq, col, NODE_NUM), axis=1, keepdims=True)
        onehot = col == sel
        msk = jnp.logical_or(msk, onehot)
        cur = jnp.where(onehot, NEG_INF, cur)
    msk = jnp.logical_or(msk, row == col)
    mask_ref[...] = msk.astype(jnp.float32)


def _fwd_kernel(data_ref, mask_ref, emb_ref, lin_wT_ref, att_i_ref, att_j_ref,
                att_em_i_ref, att_em_j_ref, gl_bias_ref, bn1_g_ref, bn1_b_ref,
                bno_g_ref, bno_b_ref, w1T_ref, b1_ref, bn2_g_ref, bn2_b_ref,
                w2_ref, b2_ref, out_ref):
    d = data_ref[0]
    emb = emb_ref[...]
    xl = jnp.dot(d, lin_wT_ref[...], preferred_element_type=jnp.float32)

    a = (jnp.sum(xl * att_i_ref[...], axis=1, keepdims=True)
         + jnp.sum(emb * att_em_i_ref[...], axis=1, keepdims=True))
    b = (jnp.sum(xl * att_j_ref[...], axis=1, keepdims=True)
         + jnp.sum(emb * att_em_j_ref[...], axis=1, keepdims=True))

    alpha = a + b.T
    alpha = jnp.where(alpha >= 0, alpha, 0.2 * alpha)
    valid = mask_ref[...] > 0.0
    am = jnp.where(valid, alpha, NEG_INF)
    amax = jnp.max(am, axis=1, keepdims=True)
    p = jnp.exp(am - amax)
    att = p / (jnp.sum(p, axis=1, keepdims=True) + 1e-16)

    agg = jnp.dot(att, xl, preferred_element_type=jnp.float32)
    out = agg + gl_bias_ref[...]
    out = out * (bn1_g_ref[...] * jax.lax.rsqrt(1.0 + EPS)) + bn1_b_ref[...]
    out = jnp.maximum(out, 0.0)

    xo = out * emb
    xo = xo * (bno_g_ref[...] * jax.lax.rsqrt(1.0 + EPS)) + bno_b_ref[...]
    xo = jnp.maximum(xo, 0.0)

    h = jnp.dot(xo, w1T_ref[...], preferred_element_type=jnp.float32) + b1_ref[...]
    h = h * (bn2_g_ref[...] * jax.lax.rsqrt(1.0 + EPS)) + bn2_b_ref[...]
    h = jnp.maximum(h, 0.0)

    y = jax.lax.dot_general(w2_ref[...], h, (((1,), (1,)), ((), ())),
                            preferred_element_type=jnp.float32)
    out_ref[0] = y + b2_ref[...]


def kernel(data, emb, lin_w, att_i, att_j, att_em_i, att_em_j, gl_bias,
           bn1_g, bn1_b, bno_g, bno_b, w1, b1, bn2_g, bn2_b, w2, b2):
    mask = pl.pallas_call(
        _mask_kernel,
        out_shape=jax.ShapeDtypeStruct((NODE_NUM, NODE_NUM), jnp.float32),
    )(emb)

    data3 = data.reshape(BATCH, NODE_NUM, INPUT_DIM)
    data3 = jnp.pad(data3, ((0, 0), (0, 0), (0, 8 - INPUT_DIM)))
    lin_wT = jnp.pad(lin_w.T, ((0, 8 - INPUT_DIM), (0, 0)))

    row = lambda v: v.reshape(1, -1)
    grid_spec = pl.GridSpec(
        grid=(BATCH,),
        in_specs=[
            pl.BlockSpec((1, NODE_NUM, 8), lambda b: (b, 0, 0)),
            pl.BlockSpec((NODE_NUM, NODE_NUM), lambda b: (0, 0)),
            pl.BlockSpec((NODE_NUM, DIM), lambda b: (0, 0)),
            pl.BlockSpec((8, DIM), lambda b: (0, 0)),
        ] + [pl.BlockSpec((1, DIM), lambda b: (0, 0))] * 9 + [
            pl.BlockSpec((DIM, INTER), lambda b: (0, 0)),
            pl.BlockSpec((1, INTER), lambda b: (0, 0)),
            pl.BlockSpec((1, INTER), lambda b: (0, 0)),
            pl.BlockSpec((1, INTER), lambda b: (0, 0)),
            pl.BlockSpec((1, INTER), lambda b: (0, 0)),
            pl.BlockSpec((1, 1), lambda b: (0, 0)),
        ],
        out_specs=pl.BlockSpec((1, 1, NODE_NUM), lambda b: (b, 0, 0)),
    )
    out = pl.pallas_call(
        _fwd_kernel,
        grid_spec=grid_spec,
        out_shape=jax.ShapeDtypeStruct((BATCH, 1, NODE_NUM), jnp.float32),
        compiler_params=pltpu.CompilerParams(
            dimension_semantics=("arbitrary",),
        ),
    )(data3, mask, emb, lin_wT, row(att_i), row(att_j), row(att_em_i),
      row(att_em_j), row(gl_bias), row(bn1_g), row(bn1_b), row(bno_g),
      row(bno_b), w1.T, row(b1), row(bn2_g), row(bn2_b), w2, b2.reshape(1, 1))
    return out.reshape(BATCH, NODE_NUM)
